# Initial kernel scaffold; baseline (speedup 1.0000x reference)
import jax
import jax.numpy as jnp
from jax.experimental import pallas as pl


def kernel(x, emb, w1, b1, w2, b2, w3, b3, dw1, db1, dw2, db2, dw3, db3, wo, bo):
    raise NotImplementedError("write your pallas kernel here")


# speedup vs baseline: 1.0416x; 1.0416x over previous
import jax
import jax.numpy as jnp
from jax import lax
from jax.experimental import pallas as pl
from jax.experimental.pallas import tpu as pltpu


def _conv3d(x, w, b, stride, padding):
    y = lax.conv_general_dilated(
        x, w, (stride, stride, stride), [(padding, padding)] * 3,
        dimension_numbers=('NCDHW', 'OIDHW', 'NCDHW'))
    return y + b.reshape(1, -1, 1, 1, 1)


def _conv_transpose3d(x, w, b, stride, padding):
    k = w.shape[2]
    w_t = jnp.transpose(jnp.flip(w, axis=(2, 3, 4)), (1, 0, 2, 3, 4))
    pad = k - 1 - padding
    y = lax.conv_general_dilated(
        x, w_t, (1, 1, 1), [(pad, pad)] * 3,
        lhs_dilation=(stride, stride, stride),
        dimension_numbers=('NCDHW', 'OIDHW', 'NCDHW'))
    return y + b.reshape(1, -1, 1, 1, 1)


def _vq_body(flat_ref, emb_ref, quant_ref, loss_ref):
    flat = flat_ref[...]
    emb = emb_ref[...]
    f2 = jnp.sum(flat * flat, axis=1, keepdims=True)
    e2 = jnp.sum(emb * emb, axis=1)
    xe = jnp.dot(flat, emb.T, preferred_element_type=jnp.float32)
    dist = f2 + e2[None, :] - 2.0 * xe
    minv = jnp.min(dist, axis=1, keepdims=True)
    K = emb.shape[0]
    iota = lax.broadcasted_iota(jnp.int32, dist.shape, 1)
    idx = jnp.min(jnp.where(dist <= minv, iota, K), axis=1)
    onehot = (iota == idx[:, None]).astype(jnp.float32)
    quant = jnp.dot(onehot, emb, preferred_element_type=jnp.float32)
    quant_ref[...] = quant
    d = quant - flat
    loss_ref[...] = jnp.reshape(1.25 * jnp.sum(d * d) / d.size, (1, 1))


def _vq(flat, emb):
    M, D = flat.shape
    quant, loss = pl.pallas_call(
        _vq_body,
        out_shape=[
            jax.ShapeDtypeStruct((M, D), jnp.float32),
            jax.ShapeDtypeStruct((1, 1), jnp.float32),
        ],
    )(flat, emb)
    return quant, loss[0, 0]


def kernel(x, emb, w1, b1, w2, b2, w3, b3, dw1, db1, dw2, db2, dw3, db3, wo, bo):
    h = jax.nn.relu(_conv3d(x, w1, b1, 2, 1))
    h = jax.nn.relu(_conv3d(h, w2, b2, 2, 1))
    h = _conv3d(h, w3, b3, 2, 1)
    h = jnp.transpose(h, (0, 2, 3, 4, 1))
    input_shape = h.shape
    ed = emb.shape[1]
    flat = h.reshape(-1, ed)
    quant, eq_loss = _vq(flat, emb)
    quant = quant.reshape(input_shape)
    quant = jnp.transpose(quant, (0, 4, 1, 2, 3))
    r = jax.nn.relu(_conv_transpose3d(quant, dw1, db1, 2, 1))
    r = jax.nn.relu(_conv_transpose3d(r, dw2, db2, 2, 1))
    r = _conv_transpose3d(r, dw3, db3, 2, 1)
    r = _conv3d(r, wo, bo, 1, 0)
    return (eq_loss, r)


# speedup vs baseline: 1.1999x; 1.1520x over previous
import jax
import jax.numpy as jnp
from jax import lax
from jax.experimental import pallas as pl
from jax.experimental.pallas import tpu as pltpu


def _conv3d(x, w, b, stride, padding):
    y = lax.conv_general_dilated(
        x, w, (stride, stride, stride), [(padding, padding)] * 3,
        dimension_numbers=('NCDHW', 'OIDHW', 'NCDHW'))
    return y + b.reshape(1, -1, 1, 1, 1)


def _conv_transpose3d(x, w, b, stride, padding):
    k = w.shape[2]
    w_t = jnp.transpose(jnp.flip(w, axis=(2, 3, 4)), (1, 0, 2, 3, 4))
    pad = k - 1 - padding
    y = lax.conv_general_dilated(
        x, w_t, (1, 1, 1), [(pad, pad)] * 3,
        lhs_dilation=(stride, stride, stride),
        dimension_numbers=('NCDHW', 'OIDHW', 'NCDHW'))
    return y + b.reshape(1, -1, 1, 1, 1)


def _vq_body(flat_ref, emb_ref, quant_ref, loss_ref):
    flat = flat_ref[...]
    emb = emb_ref[...]
    f2 = jnp.sum(flat * flat, axis=1, keepdims=True)
    e2 = jnp.sum(emb * emb, axis=1)
    xe = jnp.dot(flat, emb.T, preferred_element_type=jnp.float32)
    dist = f2 + e2[None, :] - 2.0 * xe
    minv = jnp.min(dist, axis=1, keepdims=True)
    K = emb.shape[0]
    iota = lax.broadcasted_iota(jnp.int32, dist.shape, 1)
    idx = jnp.min(jnp.where(dist <= minv, iota, K), axis=1)
    onehot = (iota == idx[:, None]).astype(jnp.float32)
    quant = jnp.dot(onehot, emb, preferred_element_type=jnp.float32)
    quant_ref[...] = quant
    d = quant - flat
    loss_ref[...] = jnp.reshape(1.25 * jnp.sum(d * d) / d.size, (1, 1))


def _vq(flat, emb):
    M, D = flat.shape
    quant, loss = pl.pallas_call(
        _vq_body,
        out_shape=[
            jax.ShapeDtypeStruct((M, D), jnp.float32),
            jax.ShapeDtypeStruct((1, 1), jnp.float32),
        ],
    )(flat, emb)
    return quant, loss[0, 0]


def kernel(x, emb, w1, b1, w2, b2, w3, b3, dw1, db1, dw2, db2, dw3, db3, wo, bo):
    h = jax.nn.relu(_conv3d(x, w1, b1, 2, 1))
    h = jax.nn.relu(_conv3d(h, w2, b2, 2, 1))
    h = _conv3d(h, w3, b3, 2, 1)
    h = jnp.transpose(h, (0, 2, 3, 4, 1))
    input_shape = h.shape
    ed = emb.shape[1]
    flat = h.reshape(-1, ed)
    quant, eq_loss = _vq(flat, emb)
    quant = quant.reshape(input_shape)
    quant = jnp.transpose(quant, (0, 4, 1, 2, 3))
    r = jax.nn.relu(_conv_transpose3d(quant, dw1, db1, 2, 1))
    r = jax.nn.relu(_conv_transpose3d(r, dw2, db2, 2, 1))
    wov = wo[0, :, 0, 0, 0]
    dw3f = jnp.einsum('icdhw,c->idhw', dw3, wov)[:, None]
    db3f = (jnp.dot(wov, db3) + bo[0]).reshape(1)
    r = _conv_transpose3d(r, dw3f, db3f, 2, 1)
    return (eq_loss, r)
